# baseline (device time: 95443 ns/iter reference)
import jax
import jax.numpy as jnp
from jax import lax
from jax.experimental import pallas as pl
from jax.experimental.pallas import tpu as pltpu

N_DEV = 4


def kernel(x, w_mat):
    m, k = x.shape
    _, n = w_mat.shape
    hm = m // 2
    qm = m // 4
    hn = n // 2
    sn = n // 4

    def body(x_ref, w_ref, out_ref,
             xst, wst, xbf, wbf,
             s1A, s1B, r1A, r1B, s2A, s2B, r2A, r2B, agA, agB,
             sem_s, sem_r, sem_in, readyY):
        r4A, r4B = s1A, s1B

        my = lax.axis_index("i")
        pY = my ^ 1
        pX = 3 - my

        barrier_sem = pltpu.get_barrier_semaphore()
        pl.semaphore_signal(
            barrier_sem, inc=1,
            device_id=(pX,), device_id_type=pl.DeviceIdType.MESH,
        )
        pl.semaphore_signal(
            readyY, inc=1,
            device_id=(pY,), device_id_type=pl.DeviceIdType.MESH,
        )

        hA = my // 2
        relA = my % 2
        qB = jnp.where(my == 0, 0, jnp.where(my == 1, 2, jnp.where(my == 2, 3, 1)))
        qBx = jnp.where(my == 0, 1, jnp.where(my == 1, 3, jnp.where(my == 2, 2, 0)))
        hB = jnp.where((my == 1) | (my == 2), 1, 0)
        relB = qB - 2 * hB

        cpx = pltpu.make_async_copy(x_ref, xst, sem_in.at[0])
        cpw = pltpu.make_async_copy(w_ref, wst, sem_in.at[1])
        cpx.start()
        cpw.start()
        cpx.wait()
        cpw.wait()
        xbf[:, :] = xst[:, :].astype(jnp.bfloat16)
        wbf[:, :] = wst[:, :].astype(jnp.bfloat16)

        strips = [
            (0, pl.ds(0, sn), pl.ds(0, sn),
             s1A, r1A, s2A, r2A, agA, r4A, hA, relA, my, pY, pX, pY),
            (4, pl.ds(sn, sn), pl.ds(sn, sn),
             s1A, r1A, s2A, r2A, agA, r4A, hA, relA, my, pY, pX, pY),
            (8, pl.ds(hn, sn), pl.ds(0, sn),
             s1B, r1B, s2B, r2B, agB, r4B, hB, relB, qB, qBx, pY, pX),
            (12, pl.ds(hn + sn, sn), pl.ds(sn, sn),
             s1B, r1B, s2B, r2B, agB, r4B, hB, relB, qB, qBx, pY, pX),
        ]
        order = (0, 2, 1, 3)

        def rdma(src, dst, sem, dev):
            return pltpu.make_async_remote_copy(
                src_ref=src, dst_ref=dst,
                send_sem=sem_s.at[sem], recv_sem=sem_r.at[sem],
                device_id=(dev,), device_id_type=pl.DeviceIdType.MESH,
            )

        ds_ = pl.ds
        rs1 = {}
        waited = {pX_key: False for pX_key in ("x", "y")}
        for i in order:
            sem, c, cs, s1, r1, _, _, _, _, h, _, _, _, d1, _ = strips[i]
            s1[:, cs] = jnp.dot(
                xbf[ds_((1 - h) * hm, hm), :], wbf[:, c],
                preferred_element_type=jnp.float32,
            ).astype(jnp.bfloat16)
            link = "x" if i < 2 else "y"
            if not waited[link]:
                if link == "x":
                    pl.semaphore_wait(barrier_sem, 1)
                else:
                    pl.semaphore_wait(readyY, 1)
                waited[link] = True
            rs1[i] = rdma(s1.at[:, cs], r1.at[:, cs], sem + 0, d1)
            rs1[i].start()

        out_ref[ds_(hA * hm, hm), ds_(0, hn)] = jnp.dot(
            xbf[ds_(hA * hm, hm), :], wbf[:, ds_(0, hn)],
            preferred_element_type=jnp.float32,
        )
        out_ref[ds_(hB * hm, hm), ds_(hn, hn)] = jnp.dot(
            xbf[ds_(hB * hm, hm), :], wbf[:, ds_(hn, hn)],
            preferred_element_type=jnp.float32,
        )

        rs2 = {}
        for i in order:
            sem, c, cs, _, r1, s2, r2, _, _, h, rel, _, pq, _, d2 = strips[i]
            rs1[i].wait_recv()
            s2[:, cs] = (
                out_ref[ds_(pq * qm, qm), c]
                + r1[ds_((1 - rel) * qm, qm), cs].astype(jnp.float32)
            ).astype(jnp.bfloat16)
            rs2[i] = rdma(s2.at[:, cs], r2.at[:, cs], sem + 1, d2)
            rs2[i].start()
        for i in order:
            _, c, cs, _, r1, _, _, _, _, _, rel, q, _, _, _ = strips[i]
            out_ref[ds_(q * qm, qm), c] = (
                out_ref[ds_(q * qm, qm), c]
                + r1[ds_(rel * qm, qm), cs].astype(jnp.float32)
            )

        ag1 = {}
        for i in order:
            sem, c, cs, _, _, _, r2, ag, _, _, rel, q, _, _, d2 = strips[i]
            rs2[i].wait_recv()
            qv = jnp.maximum(
                out_ref[ds_(q * qm, qm), c] + r2[:, cs].astype(jnp.float32),
                0.0,
            )
            out_ref[ds_(q * qm, qm), c] = qv
            ag[ds_(rel * qm, qm), cs] = qv.astype(jnp.bfloat16)
            ag1[i] = rdma(
                ag.at[ds_(rel * qm, qm), cs], ag.at[ds_(rel * qm, qm), cs],
                sem + 2, d2,
            )
            ag1[i].start()

        ag2 = {}
        for i in order:
            sem, c, cs, _, _, _, _, ag, r4, _, rel, _, pq, d1, _ = strips[i]
            ag1[i].wait_recv()
            ag2[i] = rdma(ag.at[:, cs], r4.at[:, cs], sem + 3, d1)
            ag2[i].start()
            out_ref[ds_(pq * qm, qm), c] = (
                ag[ds_((1 - rel) * qm, qm), cs].astype(jnp.float32)
            )

        for i in order:
            _, c, cs, _, _, _, _, _, r4, h, _, _, _, _, _ = strips[i]
            ag2[i].wait_recv()
            out_ref[ds_((1 - h) * hm, hm), c] = r4[:, cs].astype(jnp.float32)
        for i in order:
            for d in (rs1, rs2, ag1, ag2):
                d[i].wait_send()

    bf = jnp.bfloat16
    return pl.pallas_call(
        body,
        out_shape=jax.ShapeDtypeStruct((m, n), jnp.float32),
        in_specs=[
            pl.BlockSpec(memory_space=pl.ANY),
            pl.BlockSpec(memory_space=pl.ANY),
        ],
        out_specs=pl.BlockSpec(memory_space=pltpu.VMEM),
        scratch_shapes=[
            pltpu.VMEM((m, k), jnp.float32),
            pltpu.VMEM((k, n), jnp.float32),
            pltpu.VMEM((m, k), bf),
            pltpu.VMEM((k, n), bf),
            pltpu.VMEM((hm, hn), bf),
            pltpu.VMEM((hm, hn), bf),
            pltpu.VMEM((hm, hn), bf),
            pltpu.VMEM((hm, hn), bf),
            pltpu.VMEM((qm, hn), bf),
            pltpu.VMEM((qm, hn), bf),
            pltpu.VMEM((qm, hn), bf),
            pltpu.VMEM((qm, hn), bf),
            pltpu.VMEM((hm, hn), bf),
            pltpu.VMEM((hm, hn), bf),
            pltpu.SemaphoreType.DMA((16,)),
            pltpu.SemaphoreType.DMA((16,)),
            pltpu.SemaphoreType.DMA((2,)),
            pltpu.SemaphoreType.REGULAR,
        ],
        compiler_params=pltpu.CompilerParams(
            collective_id=0, vmem_limit_bytes=64 * 1024 * 1024
        ),
    )(x, w_mat)


# device time: 91148 ns/iter; 1.0471x vs baseline; 1.0471x over previous
import jax
import jax.numpy as jnp
from jax import lax
from jax.experimental import pallas as pl
from jax.experimental.pallas import tpu as pltpu

N_DEV = 4


def kernel(x, w_mat):
    m, k = x.shape
    _, n = w_mat.shape
    hm = m // 2
    qm = m // 4
    hn = n // 2
    sn = n // 4

    def body(x_ref, w_ref, out_ref,
             acc, xst, wst, xbf, wbf,
             s1A, s1B, r1A, r1B, s2A, s2B, r2A, r2B, agA, agB,
             sem_s, sem_r, sem_in, sem_out, readyY):
        r4A, r4B = s1A, s1B

        my = lax.axis_index("i")
        pY = my ^ 1
        pX = 3 - my

        barrier_sem = pltpu.get_barrier_semaphore()
        pl.semaphore_signal(
            barrier_sem, inc=1,
            device_id=(pX,), device_id_type=pl.DeviceIdType.MESH,
        )
        pl.semaphore_signal(
            readyY, inc=1,
            device_id=(pY,), device_id_type=pl.DeviceIdType.MESH,
        )

        hA = my // 2
        relA = my % 2
        qB = jnp.where(my == 0, 0, jnp.where(my == 1, 2, jnp.where(my == 2, 3, 1)))
        qBx = jnp.where(my == 0, 1, jnp.where(my == 1, 3, jnp.where(my == 2, 2, 0)))
        hB = jnp.where((my == 1) | (my == 2), 1, 0)
        relB = qB - 2 * hB

        cpx = pltpu.make_async_copy(x_ref, xst, sem_in.at[0])
        cpw = pltpu.make_async_copy(w_ref, wst, sem_in.at[1])
        cpx.start()
        cpw.start()
        cpx.wait()
        cpw.wait()
        xbf[:, :] = xst[:, :].astype(jnp.bfloat16)
        wbf[:, :] = wst[:, :].astype(jnp.bfloat16)

        strips = [
            (0, pl.ds(0, sn), pl.ds(0, sn),
             s1A, r1A, s2A, r2A, agA, r4A, hA, relA, my, pY, pX, pY),
            (4, pl.ds(sn, sn), pl.ds(sn, sn),
             s1A, r1A, s2A, r2A, agA, r4A, hA, relA, my, pY, pX, pY),
            (8, pl.ds(hn, sn), pl.ds(0, sn),
             s1B, r1B, s2B, r2B, agB, r4B, hB, relB, qB, qBx, pY, pX),
            (12, pl.ds(hn + sn, sn), pl.ds(sn, sn),
             s1B, r1B, s2B, r2B, agB, r4B, hB, relB, qB, qBx, pY, pX),
        ]
        order = (0, 2, 1, 3)

        def rdma(src, dst, sem, dev):
            return pltpu.make_async_remote_copy(
                src_ref=src, dst_ref=dst,
                send_sem=sem_s.at[sem], recv_sem=sem_r.at[sem],
                device_id=(dev,), device_id_type=pl.DeviceIdType.MESH,
            )

        ds_ = pl.ds
        out_cp = []
        rs1 = {}
        waited = {pX_key: False for pX_key in ("x", "y")}
        for i in order:
            sem, c, cs, s1, r1, _, _, _, _, h, _, _, _, d1, _ = strips[i]
            s1[:, cs] = jnp.dot(
                xbf[ds_((1 - h) * hm, hm), :], wbf[:, c],
                preferred_element_type=jnp.float32,
            ).astype(jnp.bfloat16)
            link = "x" if i < 2 else "y"
            if not waited[link]:
                if link == "x":
                    pl.semaphore_wait(barrier_sem, 1)
                else:
                    pl.semaphore_wait(readyY, 1)
                waited[link] = True
            rs1[i] = rdma(s1.at[:, cs], r1.at[:, cs], sem + 0, d1)
            rs1[i].start()

        acc[ds_(hA * hm, hm), ds_(0, hn)] = jnp.dot(
            xbf[ds_(hA * hm, hm), :], wbf[:, ds_(0, hn)],
            preferred_element_type=jnp.float32,
        )
        acc[ds_(hB * hm, hm), ds_(hn, hn)] = jnp.dot(
            xbf[ds_(hB * hm, hm), :], wbf[:, ds_(hn, hn)],
            preferred_element_type=jnp.float32,
        )

        rs2 = {}
        for i in order:
            sem, c, cs, _, r1, s2, r2, _, _, h, rel, _, pq, _, d2 = strips[i]
            rs1[i].wait_recv()
            s2[:, cs] = (
                acc[ds_(pq * qm, qm), c]
                + r1[ds_((1 - rel) * qm, qm), cs].astype(jnp.float32)
            ).astype(jnp.bfloat16)
            rs2[i] = rdma(s2.at[:, cs], r2.at[:, cs], sem + 1, d2)
            rs2[i].start()
        for i in order:
            _, c, cs, _, r1, _, _, _, _, _, rel, q, _, _, _ = strips[i]
            acc[ds_(q * qm, qm), c] = (
                acc[ds_(q * qm, qm), c]
                + r1[ds_(rel * qm, qm), cs].astype(jnp.float32)
            )

        ag1 = {}
        for i in order:
            sem, c, cs, _, _, _, r2, ag, _, _, rel, q, _, _, d2 = strips[i]
            rs2[i].wait_recv()
            qv = jnp.maximum(
                acc[ds_(q * qm, qm), c] + r2[:, cs].astype(jnp.float32),
                0.0,
            )
            acc[ds_(q * qm, qm), c] = qv
            ag[ds_(rel * qm, qm), cs] = qv.astype(jnp.bfloat16)
            cp = pltpu.make_async_copy(
                acc.at[ds_(q * qm, qm), c], out_ref.at[ds_(q * qm, qm), c],
                sem_out.at[3 * i + 0],
            )
            cp.start()
            out_cp.append(cp)
            ag1[i] = rdma(
                ag.at[ds_(rel * qm, qm), cs], ag.at[ds_(rel * qm, qm), cs],
                sem + 2, d2,
            )
            ag1[i].start()

        ag2 = {}
        for i in order:
            sem, c, cs, _, _, _, _, ag, r4, _, rel, _, pq, d1, _ = strips[i]
            ag1[i].wait_recv()
            ag2[i] = rdma(ag.at[:, cs], r4.at[:, cs], sem + 3, d1)
            ag2[i].start()
            acc[ds_(pq * qm, qm), c] = (
                ag[ds_((1 - rel) * qm, qm), cs].astype(jnp.float32)
            )
            cp = pltpu.make_async_copy(
                acc.at[ds_(pq * qm, qm), c], out_ref.at[ds_(pq * qm, qm), c],
                sem_out.at[3 * i + 1],
            )
            cp.start()
            out_cp.append(cp)

        for i in order:
            _, c, cs, _, _, _, _, _, r4, h, _, _, _, _, _ = strips[i]
            ag2[i].wait_recv()
            acc[ds_((1 - h) * hm, hm), c] = r4[:, cs].astype(jnp.float32)
            cp = pltpu.make_async_copy(
                acc.at[ds_((1 - h) * hm, hm), c],
                out_ref.at[ds_((1 - h) * hm, hm), c],
                sem_out.at[3 * i + 2],
            )
            cp.start()
            out_cp.append(cp)
        for cp in out_cp:
            cp.wait()
        for i in order:
            for d in (rs1, rs2, ag1, ag2):
                d[i].wait_send()

    bf = jnp.bfloat16
    return pl.pallas_call(
        body,
        out_shape=jax.ShapeDtypeStruct((m, n), jnp.float32),
        in_specs=[
            pl.BlockSpec(memory_space=pl.ANY),
            pl.BlockSpec(memory_space=pl.ANY),
        ],
        out_specs=pl.BlockSpec(memory_space=pl.ANY),
        scratch_shapes=[
            pltpu.VMEM((m, n), jnp.float32),
            pltpu.VMEM((m, k), jnp.float32),
            pltpu.VMEM((k, n), jnp.float32),
            pltpu.VMEM((m, k), bf),
            pltpu.VMEM((k, n), bf),
            pltpu.VMEM((hm, hn), bf),
            pltpu.VMEM((hm, hn), bf),
            pltpu.VMEM((hm, hn), bf),
            pltpu.VMEM((hm, hn), bf),
            pltpu.VMEM((qm, hn), bf),
            pltpu.VMEM((qm, hn), bf),
            pltpu.VMEM((qm, hn), bf),
            pltpu.VMEM((qm, hn), bf),
            pltpu.VMEM((hm, hn), bf),
            pltpu.VMEM((hm, hn), bf),
            pltpu.SemaphoreType.DMA((16,)),
            pltpu.SemaphoreType.DMA((16,)),
            pltpu.SemaphoreType.DMA((2,)),
            pltpu.SemaphoreType.DMA((12,)),
            pltpu.SemaphoreType.REGULAR,
        ],
        compiler_params=pltpu.CompilerParams(
            collective_id=0, vmem_limit_bytes=64 * 1024 * 1024
        ),
    )(x, w_mat)


# device time: 90993 ns/iter; 1.0489x vs baseline; 1.0017x over previous
import jax
import jax.numpy as jnp
from jax import lax
from jax.experimental import pallas as pl
from jax.experimental.pallas import tpu as pltpu

N_DEV = 4
NS = 4


def kernel(x, w_mat):
    m, k = x.shape
    _, n = w_mat.shape
    hm = m // 2
    qm = m // 4
    hn = n // 2
    sn = hn // NS

    def body(x_ref, w_ref, out_ref,
             acc, xst, wst, xbf, wbf,
             s1A, s1B, r1A, r1B, s2A, s2B, r2A, r2B, agA, agB,
             sem_s, sem_r, sem_in, sem_out, readyY):
        r4A, r4B = s1A, s1B

        my = lax.axis_index("i")
        pY = my ^ 1
        pX = 3 - my

        barrier_sem = pltpu.get_barrier_semaphore()
        pl.semaphore_signal(
            barrier_sem, inc=1,
            device_id=(pX,), device_id_type=pl.DeviceIdType.MESH,
        )
        pl.semaphore_signal(
            readyY, inc=1,
            device_id=(pY,), device_id_type=pl.DeviceIdType.MESH,
        )

        hA = my // 2
        relA = my % 2
        qB = jnp.where(my == 0, 0, jnp.where(my == 1, 2, jnp.where(my == 2, 3, 1)))
        qBx = jnp.where(my == 0, 1, jnp.where(my == 1, 3, jnp.where(my == 2, 2, 0)))
        hB = jnp.where((my == 1) | (my == 2), 1, 0)
        relB = qB - 2 * hB

        cpx = pltpu.make_async_copy(x_ref, xst, sem_in.at[0])
        cpw = pltpu.make_async_copy(w_ref, wst, sem_in.at[1])
        cpx.start()
        cpw.start()
        cpx.wait()
        cpw.wait()
        xbf[:, :] = xst[:, :].astype(jnp.bfloat16)
        wbf[:, :] = wst[:, :].astype(jnp.bfloat16)

        strips = []
        for j in range(NS):
            strips.append((8 * j, "x", pl.ds(j * sn, sn), pl.ds(j * sn, sn),
                           s1A, r1A, s2A, r2A, agA, r4A,
                           hA, relA, my, pY, pX, pY))
            strips.append((8 * j + 4, "y",
                           pl.ds(hn + j * sn, sn), pl.ds(j * sn, sn),
                           s1B, r1B, s2B, r2B, agB, r4B,
                           hB, relB, qB, qBx, pY, pX))
        order = range(2 * NS)

        def rdma(src, dst, sem, dev):
            return pltpu.make_async_remote_copy(
                src_ref=src, dst_ref=dst,
                send_sem=sem_s.at[sem], recv_sem=sem_r.at[sem],
                device_id=(dev,), device_id_type=pl.DeviceIdType.MESH,
            )

        ds_ = pl.ds
        out_cp = []
        rs1 = {}
        waited = {"x": False, "y": False}
        for i in order:
            sem, link, c, cs, s1, r1, _, _, _, _, h, _, _, _, d1, _ = strips[i]
            s1[:, cs] = jnp.dot(
                xbf[ds_((1 - h) * hm, hm), :], wbf[:, c],
                preferred_element_type=jnp.float32,
            ).astype(jnp.bfloat16)
            if not waited[link]:
                if link == "x":
                    pl.semaphore_wait(barrier_sem, 1)
                else:
                    pl.semaphore_wait(readyY, 1)
                waited[link] = True
            rs1[i] = rdma(s1.at[:, cs], r1.at[:, cs], sem + 0, d1)
            rs1[i].start()

        acc[ds_(hA * hm, hm), ds_(0, hn)] = jnp.dot(
            xbf[ds_(hA * hm, hm), :], wbf[:, ds_(0, hn)],
            preferred_element_type=jnp.float32,
        )
        acc[ds_(hB * hm, hm), ds_(hn, hn)] = jnp.dot(
            xbf[ds_(hB * hm, hm), :], wbf[:, ds_(hn, hn)],
            preferred_element_type=jnp.float32,
        )

        rs2 = {}
        for i in order:
            sem, _, c, cs, _, r1, s2, r2, _, _, h, rel, _, pq, _, d2 = strips[i]
            rs1[i].wait_recv()
            s2[:, cs] = (
                acc[ds_(pq * qm, qm), c]
                + r1[ds_((1 - rel) * qm, qm), cs].astype(jnp.float32)
            ).astype(jnp.bfloat16)
            rs2[i] = rdma(s2.at[:, cs], r2.at[:, cs], sem + 1, d2)
            rs2[i].start()
        for i in order:
            _, _, c, cs, _, r1, _, _, _, _, _, rel, q, _, _, _ = strips[i]
            acc[ds_(q * qm, qm), c] = (
                acc[ds_(q * qm, qm), c]
                + r1[ds_(rel * qm, qm), cs].astype(jnp.float32)
            )

        ag1 = {}
        for i in order:
            sem, _, c, cs, _, _, _, r2, ag, _, _, rel, q, _, _, d2 = strips[i]
            rs2[i].wait_recv()
            qv = jnp.maximum(
                acc[ds_(q * qm, qm), c] + r2[:, cs].astype(jnp.float32),
                0.0,
            )
            acc[ds_(q * qm, qm), c] = qv
            ag[ds_(rel * qm, qm), cs] = qv.astype(jnp.bfloat16)
            ag1[i] = rdma(
                ag.at[ds_(rel * qm, qm), cs], ag.at[ds_(rel * qm, qm), cs],
                sem + 2, d2,
            )
            ag1[i].start()
            cp = pltpu.make_async_copy(
                acc.at[ds_(q * qm, qm), c], out_ref.at[ds_(q * qm, qm), c],
                sem_out.at[3 * i + 0],
            )
            cp.start()
            out_cp.append(cp)

        ag2 = {}
        for i in order:
            sem, _, c, cs, _, _, _, _, ag, r4, _, rel, _, pq, d1, _ = strips[i]
            ag1[i].wait_recv()
            ag2[i] = rdma(ag.at[:, cs], r4.at[:, cs], sem + 3, d1)
            ag2[i].start()
            acc[ds_(pq * qm, qm), c] = (
                ag[ds_((1 - rel) * qm, qm), cs].astype(jnp.float32)
            )
            cp = pltpu.make_async_copy(
                acc.at[ds_(pq * qm, qm), c], out_ref.at[ds_(pq * qm, qm), c],
                sem_out.at[3 * i + 1],
            )
            cp.start()
            out_cp.append(cp)

        for i in order:
            _, _, c, cs, _, _, _, _, _, r4, h, _, _, _, _, _ = strips[i]
            ag2[i].wait_recv()
            acc[ds_((1 - h) * hm, hm), c] = r4[:, cs].astype(jnp.float32)
            cp = pltpu.make_async_copy(
                acc.at[ds_((1 - h) * hm, hm), c],
                out_ref.at[ds_((1 - h) * hm, hm), c],
                sem_out.at[3 * i + 2],
            )
            cp.start()
            out_cp.append(cp)
        for cp in out_cp:
            cp.wait()
        for i in order:
            for d in (rs1, rs2, ag1, ag2):
                d[i].wait_send()

    bf = jnp.bfloat16
    return pl.pallas_call(
        body,
        out_shape=jax.ShapeDtypeStruct((m, n), jnp.float32),
        in_specs=[
            pl.BlockSpec(memory_space=pl.ANY),
            pl.BlockSpec(memory_space=pl.ANY),
        ],
        out_specs=pl.BlockSpec(memory_space=pl.ANY),
        scratch_shapes=[
            pltpu.VMEM((m, n), jnp.float32),
            pltpu.VMEM((m, k), jnp.float32),
            pltpu.VMEM((k, n), jnp.float32),
            pltpu.VMEM((m, k), bf),
            pltpu.VMEM((k, n), bf),
            pltpu.VMEM((hm, hn), bf),
            pltpu.VMEM((hm, hn), bf),
            pltpu.VMEM((hm, hn), bf),
            pltpu.VMEM((hm, hn), bf),
            pltpu.VMEM((qm, hn), bf),
            pltpu.VMEM((qm, hn), bf),
            pltpu.VMEM((qm, hn), bf),
            pltpu.VMEM((qm, hn), bf),
            pltpu.VMEM((hm, hn), bf),
            pltpu.VMEM((hm, hn), bf),
            pltpu.SemaphoreType.DMA((8 * NS,)),
            pltpu.SemaphoreType.DMA((8 * NS,)),
            pltpu.SemaphoreType.DMA((2,)),
            pltpu.SemaphoreType.DMA((6 * NS,)),
            pltpu.SemaphoreType.REGULAR,
        ],
        compiler_params=pltpu.CompilerParams(
            collective_id=0, vmem_limit_bytes=64 * 1024 * 1024
        ),
    )(x, w_mat)


# device time: 90864 ns/iter; 1.0504x vs baseline; 1.0014x over previous
import jax
import jax.numpy as jnp
from jax import lax
from jax.experimental import pallas as pl
from jax.experimental.pallas import tpu as pltpu

N_DEV = 4
NS = 4


def kernel(x, w_mat):
    m, k = x.shape
    _, n = w_mat.shape
    hm = m // 2
    qm = m // 4
    hn = n // 2
    sn = hn // NS

    def body(x_ref, w_ref, out_ref,
             acc, xst, wst, xbf, wbf,
             s1A, s1B, r1A, r1B, s2A, s2B, r2A, r2B, agA, agB,
             sem_s, sem_r, sem_in, sem_out, readyY):
        r4A, r4B = s1A, s1B

        my = lax.axis_index("i")
        pY = my ^ 1
        pX = 3 - my

        barrier_sem = pltpu.get_barrier_semaphore()
        pl.semaphore_signal(
            barrier_sem, inc=1,
            device_id=(pX,), device_id_type=pl.DeviceIdType.MESH,
        )
        pl.semaphore_signal(
            readyY, inc=1,
            device_id=(pY,), device_id_type=pl.DeviceIdType.MESH,
        )

        hA = my // 2
        relA = my % 2
        qB = jnp.where(my == 0, 0, jnp.where(my == 1, 2, jnp.where(my == 2, 3, 1)))
        qBx = jnp.where(my == 0, 1, jnp.where(my == 1, 3, jnp.where(my == 2, 2, 0)))
        hB = jnp.where((my == 1) | (my == 2), 1, 0)
        relB = qB - 2 * hB

        cpx = pltpu.make_async_copy(x_ref, xst, sem_in.at[0])
        cpw = pltpu.make_async_copy(w_ref, wst, sem_in.at[1])
        cpx.start()
        cpw.start()
        cpx.wait()
        cpw.wait()
        xbf[:, :] = xst[:, :].astype(jnp.bfloat16)
        wbf[:, :] = wst[:, :].astype(jnp.bfloat16)

        strips = []
        for j in range(NS):
            strips.append((10 * j, "x", pl.ds(j * sn, sn), pl.ds(j * sn, sn),
                           s1A, r1A, s2A, r2A, agA, r4A,
                           hA, relA, my, pY, pX, pY, 1 - relA))
            strips.append((10 * j + 5, "y",
                           pl.ds(hn + j * sn, sn), pl.ds(j * sn, sn),
                           s1B, r1B, s2B, r2B, agB, r4B,
                           hB, relB, qB, qBx, pY, pX, relB))
        order = range(2 * NS)

        def rdma(src, dst, sem, dev):
            return pltpu.make_async_remote_copy(
                src_ref=src, dst_ref=dst,
                send_sem=sem_s.at[sem], recv_sem=sem_r.at[sem],
                device_id=(dev,), device_id_type=pl.DeviceIdType.MESH,
            )

        ds_ = pl.ds
        out_cp = []
        rs1 = {}
        waited = {"x": False, "y": False}
        for i in order:
            sem, link, c, cs, s1, r1, _, _, _, _, h, _, _, _, d1, _, _ = strips[i]
            s1[:, cs] = jnp.dot(
                xbf[ds_((1 - h) * hm, hm), :], wbf[:, c],
                preferred_element_type=jnp.float32,
            ).astype(jnp.bfloat16)
            if not waited[link]:
                if link == "x":
                    pl.semaphore_wait(barrier_sem, 1)
                else:
                    pl.semaphore_wait(readyY, 1)
                waited[link] = True
            rs1[i] = rdma(s1.at[:, cs], r1.at[:, cs], sem + 0, d1)
            rs1[i].start()

        acc[ds_(hA * hm, hm), ds_(0, hn)] = jnp.dot(
            xbf[ds_(hA * hm, hm), :], wbf[:, ds_(0, hn)],
            preferred_element_type=jnp.float32,
        )
        acc[ds_(hB * hm, hm), ds_(hn, hn)] = jnp.dot(
            xbf[ds_(hB * hm, hm), :], wbf[:, ds_(hn, hn)],
            preferred_element_type=jnp.float32,
        )

        rs2 = {}
        for i in order:
            sem, _, c, cs, _, r1, s2, r2, _, _, h, rel, _, pq, _, d2, _ = strips[i]
            rs1[i].wait_recv()
            s2[:, cs] = (
                acc[ds_(pq * qm, qm), c]
                + r1[ds_((1 - rel) * qm, qm), cs].astype(jnp.float32)
            ).astype(jnp.bfloat16)
            rs2[i] = rdma(s2.at[:, cs], r2.at[:, cs], sem + 1, d2)
            rs2[i].start()
        for i in order:
            _, _, c, cs, _, r1, _, _, _, _, _, rel, q, _, _, _, _ = strips[i]
            acc[ds_(q * qm, qm), c] = (
                acc[ds_(q * qm, qm), c]
                + r1[ds_(rel * qm, qm), cs].astype(jnp.float32)
            )

        ag1 = {}
        ag2e = {}
        for i in order:
            sem, _, c, cs, _, _, _, r2, ag, r4, _, rel, q, _, d1, d2, _ = strips[i]
            rs2[i].wait_recv()
            qv = jnp.maximum(
                acc[ds_(q * qm, qm), c] + r2[:, cs].astype(jnp.float32),
                0.0,
            )
            acc[ds_(q * qm, qm), c] = qv
            ag[ds_(rel * qm, qm), cs] = qv.astype(jnp.bfloat16)
            ag1[i] = rdma(
                ag.at[ds_(rel * qm, qm), cs], ag.at[ds_(rel * qm, qm), cs],
                sem + 2, d2,
            )
            ag1[i].start()
            ag2e[i] = rdma(
                ag.at[ds_(rel * qm, qm), cs], r4.at[ds_(rel * qm, qm), cs],
                sem + 4, d1,
            )
            ag2e[i].start()
            cp = pltpu.make_async_copy(
                acc.at[ds_(q * qm, qm), c], out_ref.at[ds_(q * qm, qm), c],
                sem_out.at[4 * i + 0],
            )
            cp.start()
            out_cp.append(cp)

        ag2 = {}
        for i in order:
            sem, _, c, cs, _, _, _, _, ag, r4, _, rel, _, pq, d1, _, _ = strips[i]
            ag1[i].wait_recv()
            ag2[i] = rdma(
                ag.at[ds_((1 - rel) * qm, qm), cs],
                r4.at[ds_((1 - rel) * qm, qm), cs], sem + 3, d1,
            )
            ag2[i].start()
            acc[ds_(pq * qm, qm), c] = (
                ag[ds_((1 - rel) * qm, qm), cs].astype(jnp.float32)
            )
            cp = pltpu.make_async_copy(
                acc.at[ds_(pq * qm, qm), c], out_ref.at[ds_(pq * qm, qm), c],
                sem_out.at[4 * i + 1],
            )
            cp.start()
            out_cp.append(cp)

        for i in order:
            _, _, c, cs, _, _, _, _, _, r4, h, _, _, _, _, _, relP = strips[i]
            ag2e[i].wait_recv()
            re = (1 - h) * hm + relP * qm
            acc[ds_(re, qm), c] = r4[ds_(relP * qm, qm), cs].astype(jnp.float32)
            cp = pltpu.make_async_copy(
                acc.at[ds_(re, qm), c], out_ref.at[ds_(re, qm), c],
                sem_out.at[4 * i + 2],
            )
            cp.start()
            out_cp.append(cp)
        for i in order:
            _, _, c, cs, _, _, _, _, _, r4, h, _, _, _, _, _, relP = strips[i]
            ag2[i].wait_recv()
            rl = (1 - h) * hm + (1 - relP) * qm
            acc[ds_(rl, qm), c] = r4[ds_((1 - relP) * qm, qm), cs].astype(jnp.float32)
            cp = pltpu.make_async_copy(
                acc.at[ds_(rl, qm), c], out_ref.at[ds_(rl, qm), c],
                sem_out.at[4 * i + 3],
            )
            cp.start()
            out_cp.append(cp)
        for cp in out_cp:
            cp.wait()
        for i in order:
            for d in (rs1, rs2, ag1, ag2, ag2e):
                d[i].wait_send()

    bf = jnp.bfloat16
    return pl.pallas_call(
        body,
        out_shape=jax.ShapeDtypeStruct((m, n), jnp.float32),
        in_specs=[
            pl.BlockSpec(memory_space=pl.ANY),
            pl.BlockSpec(memory_space=pl.ANY),
        ],
        out_specs=pl.BlockSpec(memory_space=pl.ANY),
        scratch_shapes=[
            pltpu.VMEM((m, n), jnp.float32),
            pltpu.VMEM((m, k), jnp.float32),
            pltpu.VMEM((k, n), jnp.float32),
            pltpu.VMEM((m, k), bf),
            pltpu.VMEM((k, n), bf),
            pltpu.VMEM((hm, hn), bf),
            pltpu.VMEM((hm, hn), bf),
            pltpu.VMEM((hm, hn), bf),
            pltpu.VMEM((hm, hn), bf),
            pltpu.VMEM((qm, hn), bf),
            pltpu.VMEM((qm, hn), bf),
            pltpu.VMEM((qm, hn), bf),
            pltpu.VMEM((qm, hn), bf),
            pltpu.VMEM((hm, hn), bf),
            pltpu.VMEM((hm, hn), bf),
            pltpu.SemaphoreType.DMA((10 * NS,)),
            pltpu.SemaphoreType.DMA((10 * NS,)),
            pltpu.SemaphoreType.DMA((2,)),
            pltpu.SemaphoreType.DMA((8 * NS,)),
            pltpu.SemaphoreType.REGULAR,
        ],
        compiler_params=pltpu.CompilerParams(
            collective_id=0, vmem_limit_bytes=64 * 1024 * 1024
        ),
    )(x, w_mat)
